# Initial kernel scaffold; baseline (speedup 1.0000x reference)
import jax
import jax.numpy as jnp
from jax.experimental import pallas as pl


def kernel(R, params, edge_index):
    raise NotImplementedError("write your pallas kernel here")


# speedup vs baseline: 2.2498x; 2.2498x over previous
import functools

import numpy as np
import jax
import jax.numpy as jnp
from jax import lax
from jax.experimental import pallas as pl
from jax.experimental.pallas import tpu as pltpu
from jax.experimental.pallas import tpu_sc as plsc

B_GRAPHS = 2048
N_NODES = 10
BN = B_GRAPHS * N_NODES
E = 131072
MAX_RADIUS = 8.0
CH = 48
D_OUT = 432
D_V = 90

NC, NS = 2, 16
NW = NC * NS
EPW = E // NW
CHUNK = 128
NCHUNK = EPW // CHUNK
RPW = BN // NS

RB = 2560
EB = 4096

f32 = jnp.float32


def _silu(x):
    return x * jax.nn.sigmoid(x)


def _edge_geom(psrc, pdst):
    ev = pdst[:, 0:3] - psrc[:, 0:3]
    ss = jnp.sum(ev * ev, axis=1, keepdims=True)
    n = jnp.sqrt(ss + 1e-9)
    u = ev / n
    x, y, z = u[:, 0:1], u[:, 1:2], u[:, 2:3]
    l1n = jnp.sqrt(x * x + y * y + z * z + 1e-9)
    l1 = u / l1n
    c3 = 2.0 * z * z - x * x - y * y
    l2raw = jnp.concatenate([x * y, y * z, c3, z * x, x * x - y * y], axis=1)
    l2n = jnp.sqrt(jnp.sum(l2raw * l2raw, axis=1, keepdims=True) + 1e-9)
    sh9 = jnp.concatenate([jnp.ones_like(x), l1, l2raw / l2n], axis=1)
    ci = lax.broadcasted_iota(jnp.int32, (1, 4), 1).astype(f32) + 1.0
    diff = n * (1.0 / 1.6) - ci
    emb = jnp.exp(-diff * diff) * (2.0 / 1.12)
    return n, sh9, emb


def _fcnet(emb, W0, W1, W2, W3):
    h = _silu(emb @ W0)
    h = _silu(h @ W1)
    h = _silu(h @ W2)
    return h @ W3


def _sh_l3(G):
    ss = jnp.sum(G * G, axis=1, keepdims=True)
    n = jnp.sqrt(ss + 1e-9)
    u = G / n
    x, y, z = u[:, 0:1], u[:, 1:2], u[:, 2:3]
    l1n = jnp.sqrt(x * x + y * y + z * z + 1e-9)
    l1 = u / l1n
    c3 = 2.0 * z * z - x * x - y * y
    l2raw = jnp.concatenate([x * y, y * z, c3, z * x, x * x - y * y], axis=1)
    l2n = jnp.sqrt(jnp.sum(l2raw * l2raw, axis=1, keepdims=True) + 1e-9)
    l2 = l2raw / l2n
    x2, y2, z2 = x * x, y * y, z * z
    l3raw = jnp.concatenate([
        y * (3.0 * x2 - y2), x * y * z, y * (4.0 * z2 - x2 - y2),
        z * (2.0 * z2 - 3.0 * x2 - 3.0 * y2), x * (4.0 * z2 - x2 - y2),
        z * (x2 - y2), x * (x2 - 3.0 * y2)], axis=1)
    l3n = jnp.sqrt(jnp.sum(l3raw * l3raw, axis=1, keepdims=True) + 1e-9)
    l3 = l3raw / l3n
    return jnp.concatenate([jnp.ones_like(x), l1, l2, l3], axis=1)


def _k0_body(lin_ref, p_ref, o_ref):
    o_ref[...] = lin_ref[...] @ p_ref[...]


def _k1_body(R_ref,
             a0, a1, a2, a3, fa0, fa1, fa2, fa3,
             b0, b1, b2, b3, fb0, fb1, fb2, fb3,
             m0, m1, m2, m3,
             fin_ref, pos_ref):
    Rf = R_ref[...]
    r5 = jnp.clip(Rf[:, 4:5].astype(jnp.int32), 0, 9)
    r6 = jnp.clip(Rf[:, 5:6].astype(jnp.int32), 0, 9)
    iot = lax.broadcasted_iota(jnp.int32, (RB, 10), 1)
    oh5 = (iot == r5).astype(f32)
    oh6 = (iot == r6).astype(f32)
    h = _silu(oh5 @ a0[...] + a1[...])
    O = h @ a2[...] + a3[...]
    h = _silu(O * fa0[...] + fa1[...])
    O = h @ fa2[...] + fa3[...]
    h = _silu(oh6 @ b0[...] + b1[...])
    Bf = h @ b2[...] + b3[...]
    h = _silu(Bf * fb0[...] + fb1[...])
    Bf = h @ fb2[...] + fb3[...]
    G = jnp.concatenate([Rf[:, 0:1], O, Bf], axis=1)
    G = _silu(G @ m0[...] + m1[...]) @ m2[...] + m3[...]
    fin_ref[...] = _sh_l3(G)
    pos_ref[...] = jnp.concatenate(
        [Rf[:, 1:4], jnp.zeros((RB, 13), f32)], axis=1)


def _k2_body(fs_ref, ps_ref, pd_ref,
             c0, c1, c2, c3, wx, wsh, w2_ref, o_ref):
    _, sh9, emb = _edge_geom(ps_ref[...], pd_ref[...])
    w_conv = _fcnet(emb, c0[...], c1[...], c2[...], c3[...])
    t = (fs_ref[...] @ wx[...]) * (sh9 @ wsh[...]) * w_conv
    tp9 = t @ w2_ref[...]
    o_ref[...] = tp9


def _k3_body(p0_ref, p1_ref, tp3_ref, wq_ref, f_ref, q_ref):
    pooled = (p0_ref[...] + p1_ref[...])[:, 0:9]
    tp3 = tp3_ref[...]
    acc = pooled[:, 0:1] * (pooled @ tp3[0:9, :])
    for i in range(1, 9):
        acc = acc + pooled[:, i:i + 1] * (pooled @ tp3[i * 9:(i + 1) * 9, :])
    f_ref[...] = acc
    q = acc @ wq_ref[...]
    q_ref[...] = jnp.concatenate([q, jnp.zeros((RB, 8), f32)], axis=1)


def _k4_body(fs_ref, qd_ref, ps_ref, pd_ref,
             k0, k1, k2, k3, kwx, kwsh,
             v0, v1, v2, v3, vwx, vwsh, dw, oA_ref, oB_ref):
    elen, sh9, emb = _edge_geom(ps_ref[...], pd_ref[...])
    fsrc = fs_ref[...]
    k = (fsrc @ kwx[...]) * (sh9 @ kwsh[...]) * _fcnet(
        emb, k0[...], k1[...], k2[...], k3[...])
    v = (fsrc @ vwx[...]) * (sh9 @ vwsh[...]) * _fcnet(
        emb, v0[...], v1[...], v2[...], v3[...])
    logit = (qd_ref[:, 0:40] * k) @ dw[...]
    xarg = 10.0 * (1.0 - elen / MAX_RADIUS)
    safe = jnp.where(xarg > 0.0, xarg, 1.0)
    cutoff = jnp.where(xarg > 0.0, jnp.exp(-1.0 / safe), 0.0)
    expv = cutoff * jnp.exp(logit)
    w = jnp.sqrt(expv)
    wv = w * v
    oA_ref[...] = wv[:, 0:48]
    oB_ref[...] = jnp.concatenate(
        [wv[:, 48:90], expv, jnp.zeros((EB, 5), f32)], axis=1)


def _k5_body(pa0_ref, pa1_ref, pb0_ref, pb1_ref, o_ref):
    sa = pa0_ref[...] + pa1_ref[...]
    sb = pb0_ref[...] + pb1_ref[...]
    num = jnp.concatenate([sa, sb[:, 0:42]], axis=1)
    z = sb[:, 42:43]
    z = jnp.where(z == 0.0, 1.0, z)
    o_ref[...] = num / jnp.sqrt(z)


def _full(shape):
    nd = len(shape)
    return pl.BlockSpec(shape, lambda i: (0,) * nd)


def _rows(bs, w):
    return pl.BlockSpec((bs, w), lambda i: (i, 0))


def _mk_mesh():
    return plsc.VectorSubcoreMesh(core_axis_name="c", subcore_axis_name="s",
                                  num_cores=NC, num_subcores=NS)


def _sc_gather(widths, by_dst):
    n = len(widths)

    @functools.partial(
        pl.kernel, mesh=_mk_mesh(),
        compiler_params=pltpu.CompilerParams(use_tc_tiling_on_sc=False),
        out_type=tuple(jax.ShapeDtypeStruct((E, w), f32) for w in widths),
        scratch_types=(
            [pltpu.VMEM((NCHUNK, CHUNK), jnp.int32)] * 2 +
            [pltpu.VMEM((CHUNK, w), f32) for w in widths] +
            [pltpu.SemaphoreType.DMA]
        ))
    def g(srcR, dstR, *rest):
        tabs = rest[:n]
        outs = rest[n:2 * n]
        idxS, idxD = rest[2 * n], rest[2 * n + 1]
        bufs = rest[2 * n + 2:3 * n + 2]
        sem = rest[3 * n + 2]
        wid = lax.axis_index("s") * NC + lax.axis_index("c")
        base = wid * EPW
        pltpu.sync_copy(srcR.at[wid], idxS)
        pltpu.sync_copy(dstR.at[wid], idxD)

        def body(j, carry):
            off = base + j * CHUNK
            for t in range(n):
                idx = idxD if by_dst[t] else idxS
                pltpu.async_copy(tabs[t].at[idx.at[j]], bufs[t], sem).wait()
                pltpu.sync_copy(bufs[t], outs[t].at[pl.ds(off, CHUNK)])
            return carry

        lax.fori_loop(0, NCHUNK, body, 0)
    return g


def _sc_scatter(width):
    @functools.partial(
        pl.kernel, mesh=_mk_mesh(),
        compiler_params=pltpu.CompilerParams(use_tc_tiling_on_sc=False),
        out_type=jax.ShapeDtypeStruct((NC, BN, width), f32),
        scratch_types=[
            pltpu.VMEM_SHARED((BN, width), f32),
            pltpu.VMEM((NCHUNK, CHUNK), jnp.int32),
            pltpu.VMEM((CHUNK, width), f32),
        ])
    def s(dstR, rows_hbm, zeros_hbm, out, accum, idxD, buf):
        cid = lax.axis_index("c")
        sid = lax.axis_index("s")
        wid = sid * NC + cid
        base = wid * EPW
        pltpu.sync_copy(dstR.at[wid], idxD)

        def zbody(t, carry):
            pltpu.sync_copy(zeros_hbm,
                            accum.at[pl.ds(sid * RPW + t * CHUNK, CHUNK)])
            return carry
        lax.fori_loop(0, RPW // CHUNK, zbody, 0)
        plsc.subcore_barrier()

        def body(j, carry):
            off = base + j * CHUNK
            pltpu.sync_copy(rows_hbm.at[pl.ds(off, CHUNK)], buf)
            pltpu.sync_copy(buf, accum.at[idxD.at[j]], add=True)
            return carry
        lax.fori_loop(0, NCHUNK, body, 0)
        plsc.subcore_barrier()

        def wbody(t, carry):
            r0 = sid * RPW + t * CHUNK
            pltpu.sync_copy(accum.at[pl.ds(r0, CHUNK)],
                            out.at[cid, pl.ds(r0, CHUNK)])
            return carry
        lax.fori_loop(0, RPW // CHUNK, wbody, 0)
    return s


def _pool_mat():
    P = np.zeros((D_OUT, 16), np.float32)
    inv = 1.0 / CH
    for c in range(CH):
        P[c, 0] = inv
    for c in range(CH * 3):
        P[CH + c, 1 + (c % 3)] = inv
    for c in range(CH * 5):
        P[CH * 4 + c, 4 + (c % 5)] = inv
    P *= 1.0 / np.sqrt(E / BN)
    return P


def kernel(R, params, edge_index):
    Rf = R.reshape(BN, 6)
    srcR = edge_index[0].astype(jnp.int32).reshape(NW, NCHUNK, CHUNK)
    dstR = edge_index[1].astype(jnp.int32).reshape(NW, NCHUNK, CHUNK)

    def b2(x):
        return x.reshape(1, -1)

    W2 = pl.pallas_call(
        _k0_body,
        grid=(1,),
        in_specs=[_full((D_OUT, D_OUT)), _full((D_OUT, 16))],
        out_specs=_full((D_OUT, 16)),
        out_shape=jax.ShapeDtypeStruct((D_OUT, 16), f32),
    )(params['lin_w'], jnp.asarray(_pool_mat()))

    p = params
    oh, oh2 = p['one_hot_mlp'], p['one_hot_mlp_2']
    ft, ft2 = p['fitnet'], p['fitnet_2']
    ml = p['mlp']
    k1_in = [Rf,
             oh[0], b2(oh[1]), oh[2], b2(oh[3]),
             ft[0], b2(ft[1]), ft[2], b2(ft[3]),
             oh2[0], b2(oh2[1]), oh2[2], b2(oh2[3]),
             ft2[0], b2(ft2[1]), ft2[2], b2(ft2[3]),
             ml[0], b2(ml[1]), ml[2], b2(ml[3])]
    k1_specs = [_rows(RB, 6)] + [_full(x.shape) for x in k1_in[1:]]
    fin, posP = pl.pallas_call(
        _k1_body,
        grid=(BN // RB,),
        in_specs=k1_specs,
        out_specs=(_rows(RB, 16), _rows(RB, 16)),
        out_shape=(jax.ShapeDtypeStruct((BN, 16), f32),
                   jax.ShapeDtypeStruct((BN, 16), f32)),
    )(*k1_in)

    gPs, gPd, gFs = _sc_gather((16, 16, 16), (False, True, False))(
        srcR, dstR, posP, posP, fin)

    cf = p['conv_fc']
    k2_in = [gFs, gPs, gPd, cf[0], cf[1], cf[2], cf[3],
             p['conv_wx'], p['conv_wsh'], W2]
    k2_specs = ([_rows(EB, 16)] * 3 +
                [_full(x.shape) for x in k2_in[3:]])
    tp9E = pl.pallas_call(
        _k2_body,
        grid=(E // EB,),
        in_specs=k2_specs,
        out_specs=_rows(EB, 16),
        out_shape=jax.ShapeDtypeStruct((E, 16), f32),
    )(*k2_in)

    zeros16 = jnp.zeros((CHUNK, 16), f32)
    parts9 = _sc_scatter(16)(dstR, tp9E, zeros16)

    tp3 = p['tp3_w'].reshape(81, 16)
    fG, qP = pl.pallas_call(
        _k3_body,
        grid=(BN // RB,),
        in_specs=[_rows(RB, 16), _rows(RB, 16),
                  _full((81, 16)), _full((16, 40))],
        out_specs=(_rows(RB, 16), _rows(RB, 48)),
        out_shape=(jax.ShapeDtypeStruct((BN, 16), f32),
                   jax.ShapeDtypeStruct((BN, 48), f32)),
    )(parts9[0], parts9[1], tp3, p['wq'])

    gF2s, gQd = _sc_gather((16, 48), (False, True))(srcR, dstR, fG, qP)

    kf, vf = p['k_fc'], p['v_fc']
    k4_in = [gF2s, gQd, gPs, gPd,
             kf[0], kf[1], kf[2], kf[3], p['k_wx'], p['k_wsh'],
             vf[0], vf[1], vf[2], vf[3], p['v_wx'], p['v_wsh'],
             p['dot_w'].reshape(40, 1)]
    k4_specs = ([_rows(EB, 16), _rows(EB, 48), _rows(EB, 16),
                 _rows(EB, 16)] + [_full(x.shape) for x in k4_in[4:]])
    attA, attB = pl.pallas_call(
        _k4_body,
        grid=(E // EB,),
        in_specs=k4_specs,
        out_specs=(_rows(EB, 48), _rows(EB, 48)),
        out_shape=(jax.ShapeDtypeStruct((E, 48), f32),
                   jax.ShapeDtypeStruct((E, 48), f32)),
    )(*k4_in)

    zeros48 = jnp.zeros((CHUNK, 48), f32)
    sc48 = _sc_scatter(48)
    partsA = sc48(dstR, attA, zeros48)
    partsB = sc48(dstR, attB, zeros48)

    f_new = pl.pallas_call(
        _k5_body,
        grid=(BN // RB,),
        in_specs=[_rows(RB, 48)] * 4,
        out_specs=_rows(RB, 90),
        out_shape=jax.ShapeDtypeStruct((BN, 90), f32),
    )(partsA[0], partsA[1], partsB[0], partsB[1])

    return f_new.reshape(B_GRAPHS, N_NODES, D_V)


# speedup vs baseline: 3.2418x; 1.4409x over previous
import functools

import numpy as np
import jax
import jax.numpy as jnp
from jax import lax
from jax.experimental import pallas as pl
from jax.experimental.pallas import tpu as pltpu
from jax.experimental.pallas import tpu_sc as plsc

B_GRAPHS = 2048
N_NODES = 10
BN = B_GRAPHS * N_NODES
E = 131072
MAX_RADIUS = 8.0
CH = 48
D_OUT = 432
D_V = 90

NC, NS = 2, 16
NW = NC * NS
EPW = E // NW
CHUNK = 128
NCHUNK = EPW // CHUNK
RPW = BN // NS

RB = 2560
EB = 4096
EB2 = 2048

f32 = jnp.float32


def _silu(x):
    return x * jax.nn.sigmoid(x)


def _edge_geom(psrc, pdst):
    ev = pdst[:, 0:3] - psrc[:, 0:3]
    ss = jnp.sum(ev * ev, axis=1, keepdims=True)
    n = jnp.sqrt(ss + 1e-9)
    u = ev / n
    x, y, z = u[:, 0:1], u[:, 1:2], u[:, 2:3]
    l1n = jnp.sqrt(x * x + y * y + z * z + 1e-9)
    l1 = u / l1n
    c3 = 2.0 * z * z - x * x - y * y
    l2raw = jnp.concatenate([x * y, y * z, c3, z * x, x * x - y * y], axis=1)
    l2n = jnp.sqrt(jnp.sum(l2raw * l2raw, axis=1, keepdims=True) + 1e-9)
    sh9 = jnp.concatenate([jnp.ones_like(x), l1, l2raw / l2n], axis=1)
    ci = lax.broadcasted_iota(jnp.int32, (1, 4), 1).astype(f32) + 1.0
    diff = n * (1.0 / 1.6) - ci
    emb = jnp.exp(-diff * diff) * (2.0 / 1.12)
    return n, sh9, emb


def _fcnet(emb, W0, W1, W2, W3):
    h = _silu(emb @ W0)
    h = _silu(h @ W1)
    h = _silu(h @ W2)
    return h @ W3


def _sh_l3(G):
    ss = jnp.sum(G * G, axis=1, keepdims=True)
    n = jnp.sqrt(ss + 1e-9)
    u = G / n
    x, y, z = u[:, 0:1], u[:, 1:2], u[:, 2:3]
    l1n = jnp.sqrt(x * x + y * y + z * z + 1e-9)
    l1 = u / l1n
    c3 = 2.0 * z * z - x * x - y * y
    l2raw = jnp.concatenate([x * y, y * z, c3, z * x, x * x - y * y], axis=1)
    l2n = jnp.sqrt(jnp.sum(l2raw * l2raw, axis=1, keepdims=True) + 1e-9)
    l2 = l2raw / l2n
    x2, y2, z2 = x * x, y * y, z * z
    l3raw = jnp.concatenate([
        y * (3.0 * x2 - y2), x * y * z, y * (4.0 * z2 - x2 - y2),
        z * (2.0 * z2 - 3.0 * x2 - 3.0 * y2), x * (4.0 * z2 - x2 - y2),
        z * (x2 - y2), x * (x2 - 3.0 * y2)], axis=1)
    l3n = jnp.sqrt(jnp.sum(l3raw * l3raw, axis=1, keepdims=True) + 1e-9)
    l3 = l3raw / l3n
    return jnp.concatenate([jnp.ones_like(x), l1, l2, l3], axis=1)


def _k0_body(lin_ref, p_ref, o_ref):
    o_ref[...] = lin_ref[...] @ p_ref[...]


def _k1_body(R_ref,
             a0, a1, a2, a3, fa0, fa1, fa2, fa3,
             b0, b1, b2, b3, fb0, fb1, fb2, fb3,
             m0, m1, m2, m3,
             fin_ref, pos_ref):
    Rf = R_ref[...]
    r5 = jnp.clip(Rf[:, 4:5].astype(jnp.int32), 0, 9)
    r6 = jnp.clip(Rf[:, 5:6].astype(jnp.int32), 0, 9)
    iot = lax.broadcasted_iota(jnp.int32, (RB, 10), 1)
    oh5 = (iot == r5).astype(f32)
    oh6 = (iot == r6).astype(f32)
    h = _silu(oh5 @ a0[...] + a1[...])
    O = h @ a2[...] + a3[...]
    h = _silu(O * fa0[...] + fa1[...])
    O = h @ fa2[...] + fa3[...]
    h = _silu(oh6 @ b0[...] + b1[...])
    Bf = h @ b2[...] + b3[...]
    h = _silu(Bf * fb0[...] + fb1[...])
    Bf = h @ fb2[...] + fb3[...]
    G = jnp.concatenate([Rf[:, 0:1], O, Bf], axis=1)
    G = _silu(G @ m0[...] + m1[...]) @ m2[...] + m3[...]
    fin_ref[...] = _sh_l3(G)
    pos_ref[...] = jnp.concatenate(
        [Rf[:, 1:4], jnp.zeros((RB, 13), f32)], axis=1)


def _k2_body(fs_ref, ps_ref, pd_ref,
             c0, c1, c2, c3, wx, wsh, w2_ref,
             o_ref, geom_ref, hv_ref):
    elen, sh9, emb = _edge_geom(ps_ref[...], pd_ref[...])
    o = _fcnet(emb, c0[...], c1[...], c2[...], c3[...])
    w_conv = o[:, 0:D_OUT]
    t = (fs_ref[...] @ wx[...]) * (sh9 @ wsh[...]) * w_conv
    tp9 = t @ w2_ref[...]
    o_ref[...] = tp9
    xarg = 10.0 * (1.0 - elen / MAX_RADIUS)
    safe = jnp.where(xarg > 0.0, xarg, 1.0)
    cutoff = jnp.where(xarg > 0.0, jnp.exp(-1.0 / safe), 0.0)
    geom_ref[...] = jnp.concatenate(
        [sh9, cutoff, jnp.zeros((EB2, 6), f32)], axis=1)
    hv_ref[...] = o[:, D_OUT:D_OUT + 130]


def _k3_body(p0_ref, p1_ref, tp3_ref, wq_ref, f_ref, q_ref):
    pooled = (p0_ref[...] + p1_ref[...])[:, 0:9]
    tp3 = tp3_ref[...]
    acc = pooled[:, 0:1] * (pooled @ tp3[0:9, :])
    for i in range(1, 9):
        acc = acc + pooled[:, i:i + 1] * (pooled @ tp3[i * 9:(i + 1) * 9, :])
    f_ref[...] = acc
    q = acc @ wq_ref[...]
    q_ref[...] = jnp.concatenate([q, jnp.zeros((RB, 8), f32)], axis=1)


def _k4_body(fs_ref, qd_ref, geom_ref, hv_ref,
             kvwx, kvwsh, dw, oA_ref, oB_ref):
    sh9 = geom_ref[:, 0:9]
    cutoff = geom_ref[:, 9:10]
    hv = hv_ref[...]
    prod = (fs_ref[...] @ kvwx[...]) * (sh9 @ kvwsh[...]) * hv
    k = prod[:, 0:40]
    v = prod[:, 40:130]
    logit = (qd_ref[:, 0:40] * k) @ dw[...]
    expv = cutoff * jnp.exp(logit)
    w = jnp.sqrt(expv)
    wv = w * v
    oA_ref[...] = wv[:, 0:48]
    oB_ref[...] = jnp.concatenate(
        [wv[:, 48:90], expv, jnp.zeros((EB, 5), f32)], axis=1)


def _k5_body(pa0_ref, pa1_ref, pb0_ref, pb1_ref, o_ref):
    sa = pa0_ref[...] + pa1_ref[...]
    sb = pb0_ref[...] + pb1_ref[...]
    num = jnp.concatenate([sa, sb[:, 0:42]], axis=1)
    z = sb[:, 42:43]
    z = jnp.where(z == 0.0, 1.0, z)
    o_ref[...] = num / jnp.sqrt(z)


def _full(shape):
    nd = len(shape)
    return pl.BlockSpec(shape, lambda i: (0,) * nd)


def _rows(bs, w):
    return pl.BlockSpec((bs, w), lambda i: (i, 0))


def _mk_mesh():
    return plsc.VectorSubcoreMesh(core_axis_name="c", subcore_axis_name="s",
                                  num_cores=NC, num_subcores=NS)


def _sc_gather(widths, by_dst):
    n = len(widths)

    @functools.partial(
        pl.kernel, mesh=_mk_mesh(),
        compiler_params=pltpu.CompilerParams(use_tc_tiling_on_sc=False),
        out_type=tuple(jax.ShapeDtypeStruct((E, w), f32) for w in widths),
        scratch_types=(
            [pltpu.VMEM((NCHUNK, CHUNK), jnp.int32)] * 2 +
            [pltpu.VMEM((CHUNK, w), f32) for w in widths] +
            [pltpu.SemaphoreType.DMA]
        ))
    def g(srcR, dstR, *rest):
        tabs = rest[:n]
        outs = rest[n:2 * n]
        idxS, idxD = rest[2 * n], rest[2 * n + 1]
        bufs = rest[2 * n + 2:3 * n + 2]
        sem = rest[3 * n + 2]
        wid = lax.axis_index("s") * NC + lax.axis_index("c")
        base = wid * EPW
        pltpu.sync_copy(srcR.at[wid], idxS)
        pltpu.sync_copy(dstR.at[wid], idxD)

        def body(j, carry):
            off = base + j * CHUNK
            for t in range(n):
                idx = idxD if by_dst[t] else idxS
                pltpu.async_copy(tabs[t].at[idx.at[j]], bufs[t], sem).wait()
                pltpu.sync_copy(bufs[t], outs[t].at[pl.ds(off, CHUNK)])
            return carry

        lax.fori_loop(0, NCHUNK, body, 0)
    return g


def _sc_scatter(width):
    @functools.partial(
        pl.kernel, mesh=_mk_mesh(),
        compiler_params=pltpu.CompilerParams(use_tc_tiling_on_sc=False),
        out_type=jax.ShapeDtypeStruct((NC, BN, width), f32),
        scratch_types=[
            pltpu.VMEM_SHARED((BN, width), f32),
            pltpu.VMEM((NCHUNK, CHUNK), jnp.int32),
            pltpu.VMEM((CHUNK, width), f32),
        ])
    def s(dstR, rows_hbm, zeros_hbm, out, accum, idxD, buf):
        cid = lax.axis_index("c")
        sid = lax.axis_index("s")
        wid = sid * NC + cid
        base = wid * EPW
        pltpu.sync_copy(dstR.at[wid], idxD)

        def zbody(t, carry):
            pltpu.sync_copy(zeros_hbm,
                            accum.at[pl.ds(sid * RPW + t * CHUNK, CHUNK)])
            return carry
        lax.fori_loop(0, RPW // CHUNK, zbody, 0)
        plsc.subcore_barrier()

        def body(j, carry):
            off = base + j * CHUNK
            pltpu.sync_copy(rows_hbm.at[pl.ds(off, CHUNK)], buf)
            pltpu.sync_copy(buf, accum.at[idxD.at[j]], add=True)
            return carry
        lax.fori_loop(0, NCHUNK, body, 0)
        plsc.subcore_barrier()

        def wbody(t, carry):
            r0 = sid * RPW + t * CHUNK
            pltpu.sync_copy(accum.at[pl.ds(r0, CHUNK)],
                            out.at[cid, pl.ds(r0, CHUNK)])
            return carry
        lax.fori_loop(0, RPW // CHUNK, wbody, 0)
    return s


def _pool_mat():
    P = np.zeros((D_OUT, 16), np.float32)
    inv = 1.0 / CH
    for c in range(CH):
        P[c, 0] = inv
    for c in range(CH * 3):
        P[CH + c, 1 + (c % 3)] = inv
    for c in range(CH * 5):
        P[CH * 4 + c, 4 + (c % 5)] = inv
    P *= 1.0 / np.sqrt(E / BN)
    return P


def kernel(R, params, edge_index):
    Rf = R.reshape(BN, 6)
    srcR = edge_index[0].astype(jnp.int32).reshape(NW, NCHUNK, CHUNK)
    dstR = edge_index[1].astype(jnp.int32).reshape(NW, NCHUNK, CHUNK)

    def b2(x):
        return x.reshape(1, -1)

    W2 = pl.pallas_call(
        _k0_body,
        grid=(1,),
        in_specs=[_full((D_OUT, D_OUT)), _full((D_OUT, 16))],
        out_specs=_full((D_OUT, 16)),
        out_shape=jax.ShapeDtypeStruct((D_OUT, 16), f32),
    )(params['lin_w'], jnp.asarray(_pool_mat()))

    p = params
    oh, oh2 = p['one_hot_mlp'], p['one_hot_mlp_2']
    ft, ft2 = p['fitnet'], p['fitnet_2']
    ml = p['mlp']
    k1_in = [Rf,
             oh[0], b2(oh[1]), oh[2], b2(oh[3]),
             ft[0], b2(ft[1]), ft[2], b2(ft[3]),
             oh2[0], b2(oh2[1]), oh2[2], b2(oh2[3]),
             ft2[0], b2(ft2[1]), ft2[2], b2(ft2[3]),
             ml[0], b2(ml[1]), ml[2], b2(ml[3])]
    k1_specs = [_rows(RB, 6)] + [_full(x.shape) for x in k1_in[1:]]
    fin, posP = pl.pallas_call(
        _k1_body,
        grid=(BN // RB,),
        in_specs=k1_specs,
        out_specs=(_rows(RB, 16), _rows(RB, 16)),
        out_shape=(jax.ShapeDtypeStruct((BN, 16), f32),
                   jax.ShapeDtypeStruct((BN, 16), f32)),
    )(*k1_in)

    gPs, gPd, gFs = _sc_gather((16, 16, 16), (False, True, False))(
        srcR, dstR, posP, posP, fin)

    cf, kf, vf = p['conv_fc'], p['k_fc'], p['v_fc']

    def _bd(ms):
        rows = sum(m.shape[0] for m in ms)
        cols = [m.shape[1] for m in ms]
        out = []
        r0 = 0
        for i, m in enumerate(ms):
            out.append(jnp.concatenate(
                [jnp.zeros((m.shape[0], sum(cols[:i])), f32), m,
                 jnp.zeros((m.shape[0], sum(cols[i + 1:])), f32)], axis=1))
            r0 += m.shape[0]
        return jnp.concatenate(out, axis=0)

    W0m = jnp.concatenate([cf[0], kf[0], vf[0]], axis=1)
    W1m = _bd([cf[1], kf[1], vf[1]])
    W2m = _bd([cf[2], kf[2], vf[2]])
    W3m = _bd([cf[3], kf[3], vf[3]])
    k2_in = [gFs, gPs, gPd, W0m, W1m, W2m, W3m,
             p['conv_wx'], p['conv_wsh'], W2]
    k2_specs = ([_rows(EB2, 16)] * 3 +
                [_full(x.shape) for x in k2_in[3:]])
    tp9E, geomE, hvE = pl.pallas_call(
        _k2_body,
        grid=(E // EB2,),
        in_specs=k2_specs,
        out_specs=(_rows(EB2, 16), _rows(EB2, 16), _rows(EB2, 130)),
        out_shape=(jax.ShapeDtypeStruct((E, 16), f32),
                   jax.ShapeDtypeStruct((E, 16), f32),
                   jax.ShapeDtypeStruct((E, 130), f32)),
    )(*k2_in)

    zeros16 = jnp.zeros((CHUNK, 16), f32)
    parts9 = _sc_scatter(16)(dstR, tp9E, zeros16)

    tp3 = p['tp3_w'].reshape(81, 16)
    fG, qP = pl.pallas_call(
        _k3_body,
        grid=(BN // RB,),
        in_specs=[_rows(RB, 16), _rows(RB, 16),
                  _full((81, 16)), _full((16, 40))],
        out_specs=(_rows(RB, 16), _rows(RB, 48)),
        out_shape=(jax.ShapeDtypeStruct((BN, 16), f32),
                   jax.ShapeDtypeStruct((BN, 48), f32)),
    )(parts9[0], parts9[1], tp3, p['wq'])

    gF2s, gQd = _sc_gather((16, 48), (False, True))(srcR, dstR, fG, qP)

    kvwx = jnp.concatenate([p['k_wx'], p['v_wx']], axis=1)
    kvwsh = jnp.concatenate([p['k_wsh'], p['v_wsh']], axis=1)
    k4_in = [gF2s, gQd, geomE, hvE,
             kvwx, kvwsh, p['dot_w'].reshape(40, 1)]
    k4_specs = ([_rows(EB, 16), _rows(EB, 48), _rows(EB, 16),
                 _rows(EB, 130)] + [_full(x.shape) for x in k4_in[4:]])
    attA, attB = pl.pallas_call(
        _k4_body,
        grid=(E // EB,),
        in_specs=k4_specs,
        out_specs=(_rows(EB, 48), _rows(EB, 48)),
        out_shape=(jax.ShapeDtypeStruct((E, 48), f32),
                   jax.ShapeDtypeStruct((E, 48), f32)),
    )(*k4_in)

    zeros48 = jnp.zeros((CHUNK, 48), f32)
    sc48 = _sc_scatter(48)
    partsA = sc48(dstR, attA, zeros48)
    partsB = sc48(dstR, attB, zeros48)

    f_new = pl.pallas_call(
        _k5_body,
        grid=(BN // RB,),
        in_specs=[_rows(RB, 48)] * 4,
        out_specs=_rows(RB, 90),
        out_shape=jax.ShapeDtypeStruct((BN, 90), f32),
    )(partsA[0], partsA[1], partsB[0], partsB[1])

    return f_new.reshape(B_GRAPHS, N_NODES, D_V)
